# baseline (speedup 1.0000x reference)
import functools

import jax
import jax.numpy as jnp
from jax import lax
from jax.experimental import pallas as pl
from jax.experimental.pallas import tpu as pltpu
from jax.experimental.pallas import tpu_sc as plsc

EPS = 1e-15
D = 32
C = 20
L = 16
NC = 2
NS = 16
NW = NC * NS
CB = 32
IW = 80


HB = 256


def _sc_sums(th, tl, pc3, nc3, Wth, Wtt, Wch, Wct, B):
    nb = B // NW
    nch = HB // CB
    rpc = CB * C // IW
    rows = CB * C
    orows = CB * D // 128
    bcol = pc3.shape[2]

    mesh = plsc.VectorSubcoreMesh(core_axis_name="c", subcore_axis_name="s")

    @functools.partial(
        pl.kernel, mesh=mesh,
        compiler_params=pltpu.CompilerParams(
            use_tc_tiling_on_sc=False, needs_layout_passes=False),
        out_type=[jax.ShapeDtypeStruct((B * D // 128, 128), jnp.float32)] * 4,
        scratch_types=[
            pltpu.VMEM((C, HB), jnp.int32),
            pltpu.VMEM((C, HB), jnp.int32),
            pltpu.VMEM((C, HB), jnp.int32),
            pltpu.VMEM((C, HB), jnp.int32),
            pltpu.VMEM((rpc, IW), jnp.int32),
            pltpu.VMEM((rpc, IW), jnp.int32),
            pltpu.VMEM((rpc, IW), jnp.int32),
            pltpu.VMEM((rpc, IW), jnp.int32),
            pltpu.VMEM((CB,), jnp.int32),
            pltpu.VMEM((CB,), jnp.int32),
            pltpu.VMEM((rows, D), jnp.float32),
            pltpu.VMEM((rows, D), jnp.float32),
            pltpu.VMEM((rows, D), jnp.float32),
            pltpu.VMEM((rows, D), jnp.float32),
            pltpu.VMEM((CB, D), jnp.float32),
            pltpu.VMEM((CB, D), jnp.float32),
            pltpu.VMEM((orows, 128), jnp.float32),
            pltpu.VMEM((orows, 128), jnp.float32),
            pltpu.VMEM((orows, 128), jnp.float32),
            pltpu.VMEM((orows, 128), jnp.float32),
            pltpu.SemaphoreType.DMA,
        ],
    )
    def body(th_h, tl_h, pc_h, nc_h, wth_h, wtt_h, wch_h, wct_h,
             o_ph, o_nh, o_pt, o_nt,
             s_ph, s_nh, s_pt, s_nt,
             iph_v, inh_v, ipt_v, int_v, ith_v, itt_v,
             prh_v, nrh_v, prt_v, nrt_v, trh_v, trt_v,
             sph_v, snh_v, spt_v, snt_v, sem):
        wid = lax.axis_index("s") * NC + lax.axis_index("c")
        base_b = wid * nb
        iota = lax.iota(jnp.int32, L)

        for h in range(nb // HB):
            b0 = base_b + h * HB
            i1 = b0 // bcol
            c0 = b0 % bcol
            pltpu.sync_copy(pc_h.at[pl.ds(0, C), i1, pl.ds(c0, HB)], s_ph)
            pltpu.sync_copy(nc_h.at[pl.ds(0, C), i1, pl.ds(c0, HB)], s_nh)
            pltpu.sync_copy(pc_h.at[pl.ds(2 * C, C), i1, pl.ds(c0, HB)], s_pt)
            pltpu.sync_copy(nc_h.at[pl.ds(2 * C, C), i1, pl.ds(c0, HB)], s_nt)
            _half(h, b0, s_ph, s_nh, s_pt, s_nt,
                  th_h, tl_h, wth_h, wtt_h, wch_h, wct_h,
                  o_ph, o_nh, o_pt, o_nt,
                  iph_v, inh_v, ipt_v, int_v, ith_v, itt_v,
                  prh_v, nrh_v, prt_v, nrt_v, trh_v, trt_v,
                  sph_v, snh_v, spt_v, snt_v, sem, iota,
                  nch, rpc, rows, orows)

    def _half(h, b0, s_ph, s_nh, s_pt, s_nt,
              th_h, tl_h, wth_h, wtt_h, wch_h, wct_h,
              o_ph, o_nh, o_pt, o_nt,
              iph_v, inh_v, ipt_v, int_v, ith_v, itt_v,
              prh_v, nrh_v, prt_v, nrt_v, trh_v, trt_v,
              sph_v, snh_v, spt_v, snt_v, sem, iota,
              nch, rpc, rows, orows):
        def chunk_body(g, carry):
            gb = b0 + g * CB
            pltpu.sync_copy(th_h.at[pl.ds(gb, CB)], ith_v)
            pltpu.sync_copy(tl_h.at[pl.ds(gb, CB)], itt_v)

            def ext_body(j, carry2):
                for k in range(IW // L):
                    pv = iota + (j * IW + k * L)
                    c_v = pv % C
                    b_v = pv // C + g * CB
                    iph_v[j, pl.ds(k * L, L)] = plsc.load_gather(
                        s_ph, [c_v, b_v])
                    ipt_v[j, pl.ds(k * L, L)] = plsc.load_gather(
                        s_pt, [c_v, b_v])
                    inh_v[j, pl.ds(k * L, L)] = plsc.load_gather(
                        s_nh, [c_v, b_v])
                    int_v[j, pl.ds(k * L, L)] = plsc.load_gather(
                        s_nt, [c_v, b_v])
                return carry2

            lax.fori_loop(0, rpc, ext_body, 0)

            cps = []
            for j in range(rpc):
                o = pl.ds(j * IW, IW)
                cps.append(pltpu.async_copy(wch_h.at[iph_v.at[j]], prh_v.at[o], sem))
                cps.append(pltpu.async_copy(wch_h.at[inh_v.at[j]], nrh_v.at[o], sem))
                cps.append(pltpu.async_copy(wct_h.at[ipt_v.at[j]], prt_v.at[o], sem))
                cps.append(pltpu.async_copy(wct_h.at[int_v.at[j]], nrt_v.at[o], sem))
            cps.append(pltpu.async_copy(wth_h.at[ith_v], trh_v, sem))
            cps.append(pltpu.async_copy(wtt_h.at[itt_v], trt_v, sem))
            for cp in cps:
                cp.wait()

            def b_body(b, carry2):
                r = b * C
                ph0 = prh_v[r, pl.ds(0, L)]
                ph1 = prh_v[r, pl.ds(L, L)]
                nh0 = nrh_v[r, pl.ds(0, L)]
                nh1 = nrh_v[r, pl.ds(L, L)]
                pt0 = prt_v[r, pl.ds(0, L)]
                pt1 = prt_v[r, pl.ds(L, L)]
                nt0 = nrt_v[r, pl.ds(0, L)]
                nt1 = nrt_v[r, pl.ds(L, L)]
                for c in range(1, C):
                    ph0 = ph0 + prh_v[r + c, pl.ds(0, L)]
                    ph1 = ph1 + prh_v[r + c, pl.ds(L, L)]
                    nh0 = nh0 + nrh_v[r + c, pl.ds(0, L)]
                    nh1 = nh1 + nrh_v[r + c, pl.ds(L, L)]
                    pt0 = pt0 + prt_v[r + c, pl.ds(0, L)]
                    pt1 = pt1 + prt_v[r + c, pl.ds(L, L)]
                    nt0 = nt0 + nrt_v[r + c, pl.ds(0, L)]
                    nt1 = nt1 + nrt_v[r + c, pl.ds(L, L)]
                th0 = trh_v[b, pl.ds(0, L)]
                th1 = trh_v[b, pl.ds(L, L)]
                tt0 = trt_v[b, pl.ds(0, L)]
                tt1 = trt_v[b, pl.ds(L, L)]
                orow = b // 4
                ocol = (b % 4) * D
                sph_v[orow, pl.ds(ocol, L)] = th0 * ph0
                sph_v[orow, pl.ds(ocol + L, L)] = th1 * ph1
                snh_v[orow, pl.ds(ocol, L)] = th0 * nh0
                snh_v[orow, pl.ds(ocol + L, L)] = th1 * nh1
                spt_v[orow, pl.ds(ocol, L)] = tt0 * pt0
                spt_v[orow, pl.ds(ocol + L, L)] = tt1 * pt1
                snt_v[orow, pl.ds(ocol, L)] = tt0 * nt0
                snt_v[orow, pl.ds(ocol + L, L)] = tt1 * nt1
                return carry2

            lax.fori_loop(0, CB, b_body, 0)
            od = pl.ds(gb * D // 128, orows)
            pltpu.sync_copy(sph_v, o_ph.at[od])
            pltpu.sync_copy(snh_v, o_nh.at[od])
            pltpu.sync_copy(spt_v, o_pt.at[od])
            pltpu.sync_copy(snt_v, o_nt.at[od])
            return carry

        lax.fori_loop(0, nch, chunk_body, 0)

    return body(th, tl, pc3, nc3, Wth, Wtt, Wch, Wct)


T_BLK = 8192


def _tc_transpose_tables(wts, V):
    g = (V + T_BLK - 1) // T_BLK
    vp = g * T_BLK
    eye = None

    def body(*refs):
        ins, outs = refs[:len(wts)], refs[len(wts):]
        idn = jnp.eye(D, dtype=jnp.float32)
        for i_r, o_r in zip(ins, outs):
            o_r[...] = jax.lax.dot_general(
                i_r[...], idn, (((0,), (0,)), ((), ())),
                preferred_element_type=jnp.float32)

    outs = pl.pallas_call(
        body,
        grid=(g,),
        compiler_params=pltpu.CompilerParams(
            fuse_transposed_lhs_in_matmul=True),
        in_specs=[pl.BlockSpec((D, T_BLK), lambda j: (0, j))] * len(wts),
        out_specs=[pl.BlockSpec((T_BLK, D), lambda j: (j, 0))] * len(wts),
        out_shape=[jax.ShapeDtypeStruct((vp, D), jnp.float32)] * len(wts),
    )(*wts)
    return outs


def _softplus(z):
    return jnp.maximum(z, 0.0) + jnp.log1p(jnp.exp(-jnp.abs(z)))


def _tc_loss(ph, nh, pt, nt, n_elems):
    def body(ph_r, nh_r, pt_r, nt_r, o_r):
        sh = jnp.sum(_softplus(-(ph_r[...] + EPS))
                     + _softplus(-(1.0 - (nh_r[...] + EPS))))
        st = jnp.sum(_softplus(-(pt_r[...] + EPS))
                     + _softplus(-(1.0 - (nt_r[...] + EPS))))
        o_r[...] = ((sh + st) * (1.0 / n_elems))[None, None]

    out = pl.pallas_call(
        body, out_shape=jax.ShapeDtypeStruct((1, 1), jnp.float32),
    )(ph, nh, pt, nt)
    return out[0, 0]


def kernel(target_triples, pos_context, neg_context,
           W_target_head, W_target_tail, W_context_head, W_context_tail):
    B = target_triples.shape[0]
    V = W_target_head.shape[0]
    th = target_triples[:, 0].astype(jnp.int32)
    tl = target_triples[:, 2].astype(jnp.int32)
    pc3 = pos_context.astype(jnp.int32).transpose(2, 1, 0).reshape(
        3 * C, B // 2048, 2048)
    nc3 = neg_context.astype(jnp.int32).transpose(2, 1, 0).reshape(
        3 * C, B // 2048, 2048)

    wth, wtt, wch, wct = _tc_transpose_tables(
        [W_target_head.T, W_target_tail.T,
         W_context_head.T, W_context_tail.T], V)

    o_ph, o_nh, o_pt, o_nt = _sc_sums(th, tl, pc3, nc3, wth, wtt, wch, wct, B)

    return _tc_loss(o_ph, o_nh, o_pt, o_nt, B * D)


# speedup vs baseline: 1.2681x; 1.2681x over previous
import functools

import jax
import jax.numpy as jnp
from jax import lax
from jax.experimental import pallas as pl
from jax.experimental.pallas import tpu as pltpu
from jax.experimental.pallas import tpu_sc as plsc

EPS = 1e-15
D = 32
C = 20
L = 16
NC = 2
NS = 16
NW = NC * NS
CB = 32
IW = 80


HB = 256


def _sc_sums(th, tl, pc3, nc3, Wth, Wtt, Wch, Wct, B):
    nb = B // NW
    nch = HB // CB
    rpc = CB * C // IW
    rows = CB * C
    orows = CB * D // 128
    bcol = pc3.shape[2]

    mesh = plsc.VectorSubcoreMesh(core_axis_name="c", subcore_axis_name="s")

    @functools.partial(
        pl.kernel, mesh=mesh,
        compiler_params=pltpu.CompilerParams(
            use_tc_tiling_on_sc=False, needs_layout_passes=False),
        out_type=[jax.ShapeDtypeStruct((B * D // 128, 128), jnp.float32)] * 4,
        scratch_types=[
            pltpu.VMEM((C, HB), jnp.int32),
            pltpu.VMEM((C, HB), jnp.int32),
            pltpu.VMEM((C, HB), jnp.int32),
            pltpu.VMEM((C, HB), jnp.int32),
            pltpu.VMEM((rpc, IW), jnp.int32),
            pltpu.VMEM((rpc, IW), jnp.int32),
            pltpu.VMEM((rpc, IW), jnp.int32),
            pltpu.VMEM((rpc, IW), jnp.int32),
            pltpu.VMEM((CB,), jnp.int32),
            pltpu.VMEM((CB,), jnp.int32),
            pltpu.VMEM((rows, D), jnp.float32),
            pltpu.VMEM((rows, D), jnp.float32),
            pltpu.VMEM((rows, D), jnp.float32),
            pltpu.VMEM((rows, D), jnp.float32),
            pltpu.VMEM((CB, D), jnp.float32),
            pltpu.VMEM((CB, D), jnp.float32),
            pltpu.VMEM((orows, 128), jnp.float32),
            pltpu.VMEM((orows, 128), jnp.float32),
            pltpu.VMEM((orows, 128), jnp.float32),
            pltpu.VMEM((orows, 128), jnp.float32),
            pltpu.SemaphoreType.DMA,
        ],
    )
    def body(th_h, tl_h, pc_h, nc_h, wth_h, wtt_h, wch_h, wct_h,
             o_ph, o_nh, o_pt, o_nt,
             s_ph, s_nh, s_pt, s_nt,
             iph_v, inh_v, ipt_v, int_v, ith_v, itt_v,
             prh_v, nrh_v, prt_v, nrt_v, trh_v, trt_v,
             sph_v, snh_v, spt_v, snt_v, sem):
        wid = lax.axis_index("s") * NC + lax.axis_index("c")
        base_b = wid * nb
        iota = lax.iota(jnp.int32, L)

        for h in range(nb // HB):
            b0 = base_b + h * HB
            i1 = b0 // bcol
            c0 = b0 % bcol
            pltpu.sync_copy(pc_h.at[pl.ds(0, C), i1, pl.ds(c0, HB)], s_ph)
            pltpu.sync_copy(nc_h.at[pl.ds(0, C), i1, pl.ds(c0, HB)], s_nh)
            pltpu.sync_copy(pc_h.at[pl.ds(2 * C, C), i1, pl.ds(c0, HB)], s_pt)
            pltpu.sync_copy(nc_h.at[pl.ds(2 * C, C), i1, pl.ds(c0, HB)], s_nt)
            _half(h, b0, s_ph, s_nh, s_pt, s_nt,
                  th_h, tl_h, wth_h, wtt_h, wch_h, wct_h,
                  o_ph, o_nh, o_pt, o_nt,
                  iph_v, inh_v, ipt_v, int_v, ith_v, itt_v,
                  prh_v, nrh_v, prt_v, nrt_v, trh_v, trt_v,
                  sph_v, snh_v, spt_v, snt_v, sem, iota,
                  nch, rpc, rows, orows)

    def _half(h, b0, s_ph, s_nh, s_pt, s_nt,
              th_h, tl_h, wth_h, wtt_h, wch_h, wct_h,
              o_ph, o_nh, o_pt, o_nt,
              iph_v, inh_v, ipt_v, int_v, ith_v, itt_v,
              prh_v, nrh_v, prt_v, nrt_v, trh_v, trt_v,
              sph_v, snh_v, spt_v, snt_v, sem, iota,
              nch, rpc, rows, orows):
        def chunk_body(g, carry):
            gb = b0 + g * CB
            pltpu.sync_copy(th_h.at[pl.ds(gb, CB)], ith_v)
            pltpu.sync_copy(tl_h.at[pl.ds(gb, CB)], itt_v)

            def ext_body(j, carry2):
                for k in range(IW // L):
                    pv = iota + (j * IW + k * L)
                    c_v = pv % C
                    b_v = pv // C + g * CB
                    iph_v[j, pl.ds(k * L, L)] = plsc.load_gather(
                        s_ph, [c_v, b_v])
                    ipt_v[j, pl.ds(k * L, L)] = plsc.load_gather(
                        s_pt, [c_v, b_v])
                    inh_v[j, pl.ds(k * L, L)] = plsc.load_gather(
                        s_nh, [c_v, b_v])
                    int_v[j, pl.ds(k * L, L)] = plsc.load_gather(
                        s_nt, [c_v, b_v])
                return carry2

            lax.fori_loop(0, rpc, ext_body, 0)

            cps = []
            for j in range(rpc):
                o = pl.ds(j * IW, IW)
                cps.append(pltpu.async_copy(wch_h.at[iph_v.at[j]], prh_v.at[o], sem))
                cps.append(pltpu.async_copy(wch_h.at[inh_v.at[j]], nrh_v.at[o], sem))
                cps.append(pltpu.async_copy(wct_h.at[ipt_v.at[j]], prt_v.at[o], sem))
                cps.append(pltpu.async_copy(wct_h.at[int_v.at[j]], nrt_v.at[o], sem))
            cps.append(pltpu.async_copy(wth_h.at[ith_v], trh_v, sem))
            cps.append(pltpu.async_copy(wtt_h.at[itt_v], trt_v, sem))
            for cp in cps:
                cp.wait()

            def b_body(b, carry2):
                r = b * C
                ph0 = prh_v[r, pl.ds(0, L)]
                ph1 = prh_v[r, pl.ds(L, L)]
                nh0 = nrh_v[r, pl.ds(0, L)]
                nh1 = nrh_v[r, pl.ds(L, L)]
                pt0 = prt_v[r, pl.ds(0, L)]
                pt1 = prt_v[r, pl.ds(L, L)]
                nt0 = nrt_v[r, pl.ds(0, L)]
                nt1 = nrt_v[r, pl.ds(L, L)]
                for c in range(1, C):
                    ph0 = ph0 + prh_v[r + c, pl.ds(0, L)]
                    ph1 = ph1 + prh_v[r + c, pl.ds(L, L)]
                    nh0 = nh0 + nrh_v[r + c, pl.ds(0, L)]
                    nh1 = nh1 + nrh_v[r + c, pl.ds(L, L)]
                    pt0 = pt0 + prt_v[r + c, pl.ds(0, L)]
                    pt1 = pt1 + prt_v[r + c, pl.ds(L, L)]
                    nt0 = nt0 + nrt_v[r + c, pl.ds(0, L)]
                    nt1 = nt1 + nrt_v[r + c, pl.ds(L, L)]
                th0 = trh_v[b, pl.ds(0, L)]
                th1 = trh_v[b, pl.ds(L, L)]
                tt0 = trt_v[b, pl.ds(0, L)]
                tt1 = trt_v[b, pl.ds(L, L)]
                orow = b // 4
                ocol = (b % 4) * D
                sph_v[orow, pl.ds(ocol, L)] = th0 * ph0
                sph_v[orow, pl.ds(ocol + L, L)] = th1 * ph1
                snh_v[orow, pl.ds(ocol, L)] = th0 * nh0
                snh_v[orow, pl.ds(ocol + L, L)] = th1 * nh1
                spt_v[orow, pl.ds(ocol, L)] = tt0 * pt0
                spt_v[orow, pl.ds(ocol + L, L)] = tt1 * pt1
                snt_v[orow, pl.ds(ocol, L)] = tt0 * nt0
                snt_v[orow, pl.ds(ocol + L, L)] = tt1 * nt1
                return carry2

            lax.fori_loop(0, CB, b_body, 0)
            od = pl.ds(gb * D // 128, orows)
            pltpu.sync_copy(sph_v, o_ph.at[od])
            pltpu.sync_copy(snh_v, o_nh.at[od])
            pltpu.sync_copy(spt_v, o_pt.at[od])
            pltpu.sync_copy(snt_v, o_nt.at[od])
            return carry

        lax.fori_loop(0, nch, chunk_body, 0)

    return body(th, tl, pc3, nc3, Wth, Wtt, Wch, Wct)


def _softplus(z):
    return jnp.maximum(z, 0.0) + jnp.log1p(jnp.exp(-jnp.abs(z)))


def _tc_loss(ph, nh, pt, nt, n_elems):
    def body(ph_r, nh_r, pt_r, nt_r, o_r):
        sh = jnp.sum(_softplus(-(ph_r[...] + EPS))
                     + _softplus(-(1.0 - (nh_r[...] + EPS))))
        st = jnp.sum(_softplus(-(pt_r[...] + EPS))
                     + _softplus(-(1.0 - (nt_r[...] + EPS))))
        o_r[...] = ((sh + st) * (1.0 / n_elems))[None, None]

    out = pl.pallas_call(
        body, out_shape=jax.ShapeDtypeStruct((1, 1), jnp.float32),
    )(ph, nh, pt, nt)
    return out[0, 0]


def kernel(target_triples, pos_context, neg_context,
           W_target_head, W_target_tail, W_context_head, W_context_tail):
    B = target_triples.shape[0]
    th = target_triples[:, 0].astype(jnp.int32)
    tl = target_triples[:, 2].astype(jnp.int32)
    pc3 = pos_context.astype(jnp.int32).transpose(2, 1, 0).reshape(
        3 * C, B // 2048, 2048)
    nc3 = neg_context.astype(jnp.int32).transpose(2, 1, 0).reshape(
        3 * C, B // 2048, 2048)

    o_ph, o_nh, o_pt, o_nt = _sc_sums(
        th, tl, pc3, nc3,
        W_target_head, W_target_tail, W_context_head, W_context_tail, B)

    return _tc_loss(o_ph, o_nh, o_pt, o_nt, B * D)
